# baseline (speedup 1.0000x reference)
import jax
import jax.numpy as jnp
from jax.experimental import pallas as pl

_K = 32
_NEG = -10000.0
_INT_MIN = -2147483648


def _r16(x):
    u = jax.lax.bitcast_convert_type(x, jnp.int32)
    e = jnp.bitwise_and(jax.lax.shift_right_logical(u, 23), jnp.int32(0xFF))
    ee = jnp.maximum(e, jnp.int32(113)) + jnp.int32(13)
    abits = jnp.bitwise_or(
        jnp.bitwise_or(jax.lax.shift_left(ee, 23), jnp.int32(0x400000)),
        jnp.bitwise_and(u, jnp.int32(_INT_MIN)))
    amagic = jax.lax.bitcast_convert_type(abits, jnp.float32)
    return (x + amagic) - amagic


def _select_body(a_ref, kt_ref, q_ref, m_ref, o_ref):
    a = a_ref[...]
    kt = kt_ref[...]
    p = jax.lax.dot_general(a, kt, (((1,), (1,)), ((), ())),
                            preferred_element_type=jnp.float32)
    p16 = _r16(p)
    s16 = _r16(p16 * q_ref[0:1, :])
    s16 = jnp.where(jnp.isnan(s16), jnp.float32(0.0), s16)
    s32 = _r16(s16 + m_ref[0:1, :])
    u = jax.lax.bitcast_convert_type(s32, jnp.int32)
    e32 = jnp.bitwise_and(jax.lax.shift_right_logical(u, 23), jnp.int32(0xFF))
    m_norm = jnp.bitwise_or(
        jax.lax.shift_left(e32 - 112, 10),
        jax.lax.shift_right_logical(jnp.bitwise_and(u, jnp.int32(0x7FFFFF)), 13))
    m_sub = (jnp.abs(s32) * jnp.float32(16777216.0)).astype(jnp.int32)
    mag = jnp.where(e32 < 113, m_sub, m_norm)
    h = jnp.where(u < 0, -mag - 1, mag).astype(jnp.int16)

    def _rowcount(mask_i16):
        parts = [mask_i16[:, 128 * j:128 * (j + 1)] for j in range(16)]
        while len(parts) > 1:
            parts = [parts[i] + parts[i + 1] for i in range(0, len(parts), 2)]
        return jnp.sum(parts[0], axis=1, keepdims=True, dtype=jnp.int32)

    one16 = jnp.int16(1)
    zero16 = jnp.int16(0)

    def cnt16(t32):
        ge16 = jnp.where(h >= t32.astype(jnp.int16), one16, zero16)
        return _rowcount(ge16)

    zed = jnp.zeros((h.shape[0], 1), jnp.int32)
    t = jnp.where(cnt16(zed) >= _K, zed, zed - 32768)
    for b in range(14, -1, -1):
        t2 = t + (1 << b)
        t = jnp.where(cnt16(t2) >= _K, t2, t)
    need = _K - cnt16(t + 1)

    colv = jax.lax.broadcasted_iota(jnp.int16, h.shape, 1)
    mcol = jnp.where(h == t.astype(jnp.int16), colv, jnp.int16(4095))

    def cntlt(j32):
        lt16 = jnp.where(mcol < j32.astype(jnp.int16), one16, zero16)
        return _rowcount(lt16)

    jsel = zed
    for b in range(10, -1, -1):
        j2 = jsel + (1 << b)
        jsel = jnp.where(cntlt(j2) < need, j2, jsel)
    sel = jnp.where(h > t.astype(jnp.int16), jnp.int16(1),
                    jnp.where(mcol <= jsel.astype(jnp.int16),
                              jnp.int16(1), jnp.int16(0)))
    o_ref[...] = sel.astype(jnp.float32) * jnp.float32(-_NEG) + jnp.float32(_NEG)


def kernel(qk_, attention_mask_, bucket_size):
    del bucket_size
    qk = jax.lax.stop_gradient(qk_).astype(jnp.float16)
    B, H, S, D = qk.shape
    att = attention_mask_.astype(jnp.float16)
    M = jnp.max(jnp.linalg.norm(qk, axis=-1))
    qk_norm = qk / M
    qk_const = jnp.linalg.norm(qk_norm, axis=-1, keepdims=True)
    qk_const = jnp.sqrt(1.0 - jnp.power(qk_const, 2))
    qk_aug = jnp.concatenate([qk_norm, qk_const.astype(jnp.float16)], axis=-1)
    a = jax.random.normal(jax.random.key(42), (B, H, S, D + 1),
                          dtype=jnp.float32).astype(jnp.float16)
    q = jnp.sum(qk_aug * a, axis=-1)

    Dp = 128
    pad = Dp - (D + 1)
    a_p = jnp.pad(a, ((0, 0), (0, 0), (0, 0), (0, pad)))[0]
    a_p = a_p.reshape(H * S, Dp).astype(jnp.float32)
    kt = jnp.pad(qk_aug, ((0, 0), (0, 0), (0, 0), (0, pad)))[0]
    kt = kt.reshape(H * S, Dp).astype(jnp.float32)
    qv = jnp.broadcast_to(q[0][:, None, :], (H, 8, S))
    qv = qv.reshape(H * 8, S).astype(jnp.float32)
    mv = jnp.broadcast_to(att.reshape(1, 1, S), (1, 8, S))
    mv = mv.reshape(8, S).astype(jnp.float32)

    BX = 256
    NX = S // BX
    grid = (H, NX)
    out = pl.pallas_call(
        _select_body,
        grid=grid,
        in_specs=[
            pl.BlockSpec((BX, Dp), lambda h, x: (h * NX + x, 0)),
            pl.BlockSpec((S, Dp), lambda h, x: (h, 0)),
            pl.BlockSpec((8, S), lambda h, x: (h, 0)),
            pl.BlockSpec((8, S), lambda h, x: (0, 0)),
        ],
        out_specs=pl.BlockSpec((BX, S), lambda h, x: (h * NX + x, 0)),
        out_shape=jax.ShapeDtypeStruct((H * S, S), jnp.float32),
    )(a_p, kt, qv, mv)
    return out.reshape(1, H, S, S)


# speedup vs baseline: 1.1229x; 1.1229x over previous
import jax
import jax.numpy as jnp
from jax.experimental import pallas as pl

_K = 32
_NEG = -10000.0
_INT_MIN = -2147483648


def _r16(x):
    u = jax.lax.bitcast_convert_type(x, jnp.int32)
    e = jnp.bitwise_and(jax.lax.shift_right_logical(u, 23), jnp.int32(0xFF))
    ee = jnp.maximum(e, jnp.int32(113)) + jnp.int32(13)
    abits = jnp.bitwise_or(
        jnp.bitwise_or(jax.lax.shift_left(ee, 23), jnp.int32(0x400000)),
        jnp.bitwise_and(u, jnp.int32(_INT_MIN)))
    amagic = jax.lax.bitcast_convert_type(abits, jnp.float32)
    return (x + amagic) - amagic


def _select_body(a_ref, kt_ref, q_ref, m_ref, o_ref):
    a = a_ref[...]
    kt = kt_ref[...]
    p = jax.lax.dot_general(a, kt, (((1,), (0,)), ((), ())),
                            preferred_element_type=jnp.float32)
    p16 = _r16(p)
    s16 = _r16(p16 * q_ref[0:1, :])
    s16 = jnp.where(jnp.isnan(s16), jnp.float32(0.0), s16)
    w = s16 + m_ref[0:1, :]
    u = jax.lax.bitcast_convert_type(w, jnp.int32)
    au = jnp.bitwise_and(u, jnp.int32(0x7FFFFFFF))
    v = au - jnp.int32(0x38000000)
    lsb = jnp.bitwise_and(jax.lax.shift_right_logical(v, 13), jnp.int32(1))
    m_norm = jax.lax.shift_right_logical(v + (lsb + jnp.int32(0xFFF)), 13)
    m_sub = (jax.lax.bitcast_convert_type(au, jnp.float32)
             * jnp.float32(16777216.0)).astype(jnp.int32)
    mag = jnp.where(au < jnp.int32(0x38800000), m_sub, m_norm)
    h = jnp.where(u < 0, jnp.bitwise_xor(mag, jnp.int32(-1)),
                  mag).astype(jnp.int16)

    def _rowcount(mask_i16):
        parts = [mask_i16[:, 128 * j:128 * (j + 1)] for j in range(16)]
        while len(parts) > 1:
            parts = [parts[i] + parts[i + 1] for i in range(0, len(parts), 2)]
        return jnp.sum(parts[0], axis=1, keepdims=True, dtype=jnp.int32)

    one16 = jnp.int16(1)
    zero16 = jnp.int16(0)

    def cnt16(t32):
        ge16 = jnp.where(h >= t32.astype(jnp.int16), one16, zero16)
        return _rowcount(ge16)

    zed = jnp.zeros((h.shape[0], 1), jnp.int32)
    t = jnp.where(cnt16(zed) >= _K, zed, zed - 32768)
    for b in range(14, -1, -1):
        t2 = t + (1 << b)
        t = jnp.where(cnt16(t2) >= _K, t2, t)
    need = _K - cnt16(t + 1)

    colv = jax.lax.broadcasted_iota(jnp.int16, h.shape, 1)
    mcol = jnp.where(h == t.astype(jnp.int16), colv, jnp.int16(4095))

    def cntlt(j32):
        lt16 = jnp.where(mcol < j32.astype(jnp.int16), one16, zero16)
        return _rowcount(lt16)

    jsel = zed
    for b in range(10, -1, -1):
        j2 = jsel + (1 << b)
        jsel = jnp.where(cntlt(j2) < need, j2, jsel)
    sel = jnp.where(h > t.astype(jnp.int16), jnp.int16(1),
                    jnp.where(mcol <= jsel.astype(jnp.int16),
                              jnp.int16(1), jnp.int16(0)))
    o_ref[...] = sel.astype(jnp.float32) * jnp.float32(-_NEG) + jnp.float32(_NEG)


def kernel(qk_, attention_mask_, bucket_size):
    del bucket_size
    qk = jax.lax.stop_gradient(qk_).astype(jnp.float16)
    B, H, S, D = qk.shape
    att = attention_mask_.astype(jnp.float16)
    M = jnp.max(jnp.linalg.norm(qk, axis=-1))
    qk_norm = qk / M
    qk_const = jnp.linalg.norm(qk_norm, axis=-1, keepdims=True)
    qk_const = jnp.sqrt(1.0 - jnp.power(qk_const, 2))
    qk_aug = jnp.concatenate([qk_norm, qk_const.astype(jnp.float16)], axis=-1)
    a = jax.random.normal(jax.random.key(42), (B, H, S, D + 1),
                          dtype=jnp.float32).astype(jnp.float16)
    q = jnp.sum(qk_aug * a, axis=-1)

    Dp = 128
    pad = Dp - (D + 1)
    a_p = jnp.pad(a, ((0, 0), (0, 0), (0, 0), (0, pad)))[0]
    a_p = a_p.reshape(H * S, Dp).astype(jnp.float32)
    kt = jnp.pad(qk_aug, ((0, 0), (0, 0), (0, 0), (0, pad)))[0]
    kt = jnp.swapaxes(kt, 1, 2).reshape(H * Dp, S).astype(jnp.float32)
    qv = jnp.broadcast_to(q[0][:, None, :], (H, 8, S))
    qv = qv.reshape(H * 8, S).astype(jnp.float32)
    mv = jnp.broadcast_to(att.reshape(1, 1, S), (1, 8, S))
    mv = mv.reshape(8, S).astype(jnp.float32)

    BX = 256
    NX = S // BX
    grid = (H, NX)
    out = pl.pallas_call(
        _select_body,
        grid=grid,
        in_specs=[
            pl.BlockSpec((BX, Dp), lambda h, x: (h * NX + x, 0)),
            pl.BlockSpec((Dp, S), lambda h, x: (h, 0)),
            pl.BlockSpec((8, S), lambda h, x: (h, 0)),
            pl.BlockSpec((8, S), lambda h, x: (0, 0)),
        ],
        out_specs=pl.BlockSpec((BX, S), lambda h, x: (h * NX + x, 0)),
        out_shape=jax.ShapeDtypeStruct((H * S, S), jnp.float32),
    )(a_p, kt, qv, mv)
    return out.reshape(1, H, S, S)


# speedup vs baseline: 1.1790x; 1.0499x over previous
import jax
import jax.numpy as jnp
from jax.experimental import pallas as pl

_K = 32
_NEG = -10000.0
_INT_MIN = -2147483648


def _r16(x):
    u = jax.lax.bitcast_convert_type(x, jnp.int32)
    ebits = jnp.maximum(jnp.bitwise_and(u, jnp.int32(0x7F800000)),
                        jnp.int32(0x38800000))
    abits = jnp.bitwise_or(ebits + jnp.int32(0x06C00000),
                           jnp.bitwise_and(u, jnp.int32(_INT_MIN)))
    amagic = jax.lax.bitcast_convert_type(abits, jnp.float32)
    return (x + amagic) - amagic


def _select_body(a_ref, kt_ref, q_ref, m_ref, o_ref):
    a = a_ref[...]
    kt = kt_ref[...]
    p = jax.lax.dot_general(a, kt, (((1,), (0,)), ((), ())),
                            preferred_element_type=jnp.float32)
    p16 = _r16(p)
    s16 = _r16(p16 * q_ref[0:1, :])
    s16 = jnp.where(jnp.isnan(s16), jnp.float32(0.0), s16)
    w = s16 + m_ref[0:1, :]
    u = jax.lax.bitcast_convert_type(w, jnp.int32)
    au = jnp.bitwise_and(u, jnp.int32(0x7FFFFFFF))
    v = au - jnp.int32(0x38000000)
    lsb = jnp.bitwise_and(jax.lax.shift_right_logical(v, 13), jnp.int32(1))
    m_norm = jax.lax.shift_right_logical(v + (lsb + jnp.int32(0xFFF)), 13)
    m_sub = (jax.lax.bitcast_convert_type(au, jnp.float32)
             * jnp.float32(16777216.0)).astype(jnp.int32)
    mag = jnp.where(au < jnp.int32(0x38800000), m_sub, m_norm)
    h = jnp.where(u < 0, jnp.bitwise_xor(mag, jnp.int32(-1)),
                  mag).astype(jnp.int16)

    def _rowcount(mask_i16):
        parts = [mask_i16[:, 128 * j:128 * (j + 1)] for j in range(16)]
        while len(parts) > 1:
            parts = [parts[i] + parts[i + 1] for i in range(0, len(parts), 2)]
        return jnp.sum(parts[0], axis=1, keepdims=True, dtype=jnp.int32)

    one16 = jnp.int16(1)
    zero16 = jnp.int16(0)

    def cnt16(t32):
        ge16 = jnp.where(h >= t32.astype(jnp.int16), one16, zero16)
        return _rowcount(ge16)

    zed = jnp.zeros((h.shape[0], 1), jnp.int32)
    t = jnp.where(cnt16(zed) >= _K, zed, zed - 32768)
    for b in range(14, -1, -1):
        t2 = t + (1 << b)
        t = jnp.where(cnt16(t2) >= _K, t2, t)
    need = _K - cnt16(t + 1)

    colv = jax.lax.broadcasted_iota(jnp.int16, h.shape, 1)
    mcol = jnp.where(h == t.astype(jnp.int16), colv, jnp.int16(4095))

    def cntlt(j32):
        lt16 = jnp.where(mcol < j32.astype(jnp.int16), one16, zero16)
        return _rowcount(lt16)

    jsel = zed
    for b in range(10, -1, -1):
        j2 = jsel + (1 << b)
        jsel = jnp.where(cntlt(j2) < need, j2, jsel)
    sel = jnp.where(h > t.astype(jnp.int16), jnp.int16(1),
                    jnp.where(mcol <= jsel.astype(jnp.int16),
                              jnp.int16(1), jnp.int16(0)))
    o_ref[...] = sel.astype(jnp.float32) * jnp.float32(-_NEG) + jnp.float32(_NEG)


def kernel(qk_, attention_mask_, bucket_size):
    del bucket_size
    qk = jax.lax.stop_gradient(qk_).astype(jnp.float16)
    B, H, S, D = qk.shape
    att = attention_mask_.astype(jnp.float16)
    M = jnp.max(jnp.linalg.norm(qk, axis=-1))
    qk_norm = qk / M
    qk_const = jnp.linalg.norm(qk_norm, axis=-1, keepdims=True)
    qk_const = jnp.sqrt(1.0 - jnp.power(qk_const, 2))
    qk_aug = jnp.concatenate([qk_norm, qk_const.astype(jnp.float16)], axis=-1)
    a = jax.random.normal(jax.random.key(42), (B, H, S, D + 1),
                          dtype=jnp.float32).astype(jnp.float16)
    q = jnp.sum(qk_aug * a, axis=-1)

    Dp = 128
    pad = Dp - (D + 1)
    a_p = jnp.pad(a, ((0, 0), (0, 0), (0, 0), (0, pad)))[0]
    a_p = a_p.reshape(H * S, Dp).astype(jnp.float32)
    kt = jnp.pad(qk_aug, ((0, 0), (0, 0), (0, 0), (0, pad)))[0]
    kt = jnp.swapaxes(kt, 1, 2).reshape(H * Dp, S).astype(jnp.float32)
    qv = jnp.broadcast_to(q[0][:, None, :], (H, 8, S))
    qv = qv.reshape(H * 8, S).astype(jnp.float32)
    mv = jnp.broadcast_to(att.reshape(1, 1, S), (1, 8, S))
    mv = mv.reshape(8, S).astype(jnp.float32)

    BX = 256
    NX = S // BX
    grid = (H, NX)
    out = pl.pallas_call(
        _select_body,
        grid=grid,
        in_specs=[
            pl.BlockSpec((BX, Dp), lambda h, x: (h * NX + x, 0)),
            pl.BlockSpec((Dp, S), lambda h, x: (h, 0)),
            pl.BlockSpec((8, S), lambda h, x: (h, 0)),
            pl.BlockSpec((8, S), lambda h, x: (0, 0)),
        ],
        out_specs=pl.BlockSpec((BX, S), lambda h, x: (h * NX + x, 0)),
        out_shape=jax.ShapeDtypeStruct((H * S, S), jnp.float32),
    )(a_p, kt, qv, mv)
    return out.reshape(1, H, S, S)


# speedup vs baseline: 1.1932x; 1.0121x over previous
import jax
import jax.numpy as jnp
from jax.experimental import pallas as pl

_K = 32
_NEG = -10000.0
_INT_MIN = -2147483648


def _r16(x):
    u = jax.lax.bitcast_convert_type(x, jnp.int32)
    ebits = jnp.maximum(jnp.bitwise_and(u, jnp.int32(0x7F800000)),
                        jnp.int32(0x38800000))
    abits = jnp.bitwise_or(ebits + jnp.int32(0x06C00000),
                           jnp.bitwise_and(u, jnp.int32(_INT_MIN)))
    amagic = jax.lax.bitcast_convert_type(abits, jnp.float32)
    return (x + amagic) - amagic


def _select_body(a_ref, kt_ref, q_ref, m_ref, o_ref):
    a = a_ref[...]
    kt = kt_ref[...]
    p = jax.lax.dot_general(a, kt, (((1,), (0,)), ((), ())),
                            preferred_element_type=jnp.float32)
    p16 = _r16(p)
    s16 = _r16(p16 * q_ref[0:1, :])
    s16 = jnp.where(jnp.isnan(s16), jnp.float32(0.0), s16)
    w = s16 + m_ref[0:1, :]
    u = jax.lax.bitcast_convert_type(w, jnp.int32)
    au = jnp.bitwise_and(u, jnp.int32(0x7FFFFFFF))
    v = au - jnp.int32(0x38000000)
    lsb = jnp.bitwise_and(jax.lax.shift_right_logical(v, 13), jnp.int32(1))
    m_norm = jax.lax.shift_right_logical(v + (lsb + jnp.int32(0xFFF)), 13)
    m_sub = (jax.lax.bitcast_convert_type(au, jnp.float32)
             * jnp.float32(16777216.0)).astype(jnp.int32)
    mag = jnp.where(au < jnp.int32(0x38800000), m_sub, m_norm)
    h = jnp.where(u < 0, jnp.bitwise_xor(mag, jnp.int32(-1)),
                  mag).astype(jnp.int16)

    def _rowcount(mask_i16):
        parts = [mask_i16[:, 128 * j:128 * (j + 1)] for j in range(16)]
        while len(parts) > 1:
            parts = [parts[i] + parts[i + 1] for i in range(0, len(parts), 2)]
        return jnp.sum(parts[0], axis=1, keepdims=True, dtype=jnp.int32)

    one16 = jnp.int16(1)
    zero16 = jnp.int16(0)

    def cnt16(t32):
        ge16 = jnp.where(h >= t32.astype(jnp.int16), one16, zero16)
        return _rowcount(ge16)

    zed = jnp.zeros((h.shape[0], 1), jnp.int32)
    t = jnp.where(cnt16(zed) >= _K, zed, zed - 32768)
    for b in range(14, -1, -1):
        t2 = t + (1 << b)
        t = jnp.where(cnt16(t2) >= _K, t2, t)
    need = _K - cnt16(t + 1)

    colv = jax.lax.broadcasted_iota(jnp.int16, h.shape, 1)
    mcol = jnp.where(h == t.astype(jnp.int16), colv, jnp.int16(4095))

    def cntlt(j32):
        lt16 = jnp.where(mcol < j32.astype(jnp.int16), one16, zero16)
        return _rowcount(lt16)

    jsel = zed
    for b in range(10, -1, -1):
        j2 = jsel + (1 << b)
        jsel = jnp.where(cntlt(j2) < need, j2, jsel)
    sel = jnp.where(h > t.astype(jnp.int16), jnp.int16(1),
                    jnp.where(mcol <= jsel.astype(jnp.int16),
                              jnp.int16(1), jnp.int16(0)))
    o_ref[...] = sel.astype(jnp.float32) * jnp.float32(-_NEG) + jnp.float32(_NEG)


def kernel(qk_, attention_mask_, bucket_size):
    del bucket_size
    qk = jax.lax.stop_gradient(qk_).astype(jnp.float16)
    B, H, S, D = qk.shape
    att = attention_mask_.astype(jnp.float16)
    M = jnp.max(jnp.linalg.norm(qk, axis=-1))
    qk_norm = qk / M
    qk_const = jnp.linalg.norm(qk_norm, axis=-1, keepdims=True)
    qk_const = jnp.sqrt(1.0 - jnp.power(qk_const, 2))
    qk_aug = jnp.concatenate([qk_norm, qk_const.astype(jnp.float16)], axis=-1)
    a = jax.random.normal(jax.random.key(42), (B, H, S, D + 1),
                          dtype=jnp.float32).astype(jnp.float16)
    q = jnp.sum(qk_aug * a, axis=-1)

    Dp = 128
    pad = Dp - (D + 1)
    a_p = jnp.pad(a, ((0, 0), (0, 0), (0, 0), (0, pad)))[0]
    a_p = a_p.reshape(H * S, Dp).astype(jnp.float32)
    kt = jnp.pad(qk_aug, ((0, 0), (0, 0), (0, 0), (0, pad)))[0]
    kt = jnp.swapaxes(kt, 1, 2).reshape(H * Dp, S).astype(jnp.float32)
    qv = jnp.broadcast_to(q[0][:, None, :], (H, 8, S))
    qv = qv.reshape(H * 8, S).astype(jnp.float32)
    mv = jnp.broadcast_to(att.reshape(1, 1, S), (1, 8, S))
    mv = mv.reshape(8, S).astype(jnp.float32)

    BX = 512
    NX = S // BX
    grid = (H, NX)
    out = pl.pallas_call(
        _select_body,
        grid=grid,
        in_specs=[
            pl.BlockSpec((BX, Dp), lambda h, x: (h * NX + x, 0)),
            pl.BlockSpec((Dp, S), lambda h, x: (h, 0)),
            pl.BlockSpec((8, S), lambda h, x: (h, 0)),
            pl.BlockSpec((8, S), lambda h, x: (0, 0)),
        ],
        out_specs=pl.BlockSpec((BX, S), lambda h, x: (h * NX + x, 0)),
        out_shape=jax.ShapeDtypeStruct((H * S, S), jnp.float32),
    )(a_p, kt, qv, mv)
    return out.reshape(1, H, S, S)
